# Initial kernel scaffold; baseline (speedup 1.0000x reference)
import functools

import jax
import jax.numpy as jnp
from jax import lax
from jax.experimental import pallas as pl
from jax.experimental.pallas import tpu as pltpu
from jax.experimental.pallas import tpu_sc as plsc

N_NODES = 10000
N_EDGES = 320000
D = 128
N_CAND = 5000

NC = 2
NS = 16
NW = NC * NS

EDGES_PER_W = N_EDGES // NW
CHUNK = 80
N_CHUNKS = EDGES_PER_W // CHUNK

CAND_PAD = 5120
CAND_PER_TILE = CAND_PAD // NS
CAND_CHUNKS = CAND_PER_TILE // CHUNK


def _sc_kernel(constraint, cv_edge_index, edge_attr, cand_pad, zeros):
  mesh = plsc.VectorSubcoreMesh(core_axis_name="c", subcore_axis_name="s")

  @functools.partial(
      pl.kernel,
      mesh=mesh,
      out_type=jax.ShapeDtypeStruct((NC, CAND_PAD, D), jnp.float32),
      scratch_types=[
          pltpu.VMEM((CHUNK,), jnp.int32),
          pltpu.VMEM((CHUNK,), jnp.int32),
          pltpu.VMEM((CHUNK, 4), jnp.float32),
          pltpu.VMEM((CHUNK, D), jnp.float32),
          pltpu.VMEM_SHARED((N_NODES, D), jnp.float32),
      ],
  )
  def k(constraint_hbm, cv_hbm, ea_hbm, cand_hbm, zeros_hbm, outp_hbm,
        cidx, vidx, wbuf, rows, acc):
    c = lax.axis_index("c")
    s = lax.axis_index("s")
    wid = c * NS + s

    rows_per_tile = N_NODES // NS
    pltpu.sync_copy(zeros_hbm.at[pl.ds(s * rows_per_tile, rows_per_tile)],
                    acc.at[pl.ds(s * rows_per_tile, rows_per_tile)])
    plsc.subcore_barrier()

    ebase = wid * EDGES_PER_W

    def chunk_body(kk, _):
      base = ebase + kk * CHUNK
      pltpu.sync_copy(cv_hbm.at[0, pl.ds(base, CHUNK)], cidx)
      pltpu.sync_copy(cv_hbm.at[1, pl.ds(base, CHUNK)], vidx)
      pltpu.sync_copy(ea_hbm.at[pl.ds(base, CHUNK), :], wbuf)
      pltpu.sync_copy(constraint_hbm.at[cidx], rows)

      def scale(e, _):
        wv = plsc.load_gather(
            wbuf, [jnp.full((16,), e, jnp.int32), jnp.zeros((16,), jnp.int32)])
        for j in range(D // 16):
          rows[e, pl.ds(j * 16, 16)] = rows[e, pl.ds(j * 16, 16)] * wv
        return 0

      lax.fori_loop(0, CHUNK, scale, 0)
      pltpu.sync_copy(rows, acc.at[vidx], add=True)
      return 0

    lax.fori_loop(0, N_CHUNKS, chunk_body, 0)
    plsc.subcore_barrier()

    def cand_body(q, _):
      cbase = s * CAND_PER_TILE + q * CHUNK
      pltpu.sync_copy(cand_hbm.at[pl.ds(cbase, CHUNK)], cidx)
      pltpu.sync_copy(acc.at[cidx], rows)
      pltpu.sync_copy(rows, outp_hbm.at[c, pl.ds(cbase, CHUNK)])
      return 0

    lax.fori_loop(0, CAND_CHUNKS, cand_body, 0)

  return k(constraint, cv_edge_index, edge_attr, cand_pad, zeros)


def _combine(partials):
  def body(p_ref, o_ref):
    o_ref[...] = p_ref[0] + p_ref[1]

  blk = 256
  return pl.pallas_call(
      body,
      grid=(CAND_PAD // blk,),
      in_specs=[pl.BlockSpec((NC, blk, D), lambda i: (0, i, 0))],
      out_specs=pl.BlockSpec((blk, D), lambda i: (i, 0)),
      out_shape=jax.ShapeDtypeStruct((CAND_PAD, D), jnp.float32),
  )(partials)


def kernel(constraint, variable, cv_edge_index, edge_attr, cand_mask):
  cand_pad = jnp.concatenate(
      [cand_mask, jnp.zeros((CAND_PAD - N_CAND,), jnp.int32)])
  zeros = jnp.zeros_like(variable)
  partials = _sc_kernel(constraint, cv_edge_index, edge_attr, cand_pad, zeros)
  return _combine(partials)[:N_CAND]


# speedup vs baseline: 4.3742x; 4.3742x over previous
import functools

import jax
import jax.numpy as jnp
from jax import lax
from jax.experimental import pallas as pl
from jax.experimental.pallas import tpu as pltpu
from jax.experimental.pallas import tpu_sc as plsc

N_NODES = 10000
N_EDGES = 320000
D = 128
N_CAND = 5000

NC = 2
NS = 16
NW = NC * NS

EDGES_PER_W = N_EDGES // NW
CHUNK = 80
N_CHUNKS = EDGES_PER_W // CHUNK

CAND_PAD = 5120
CAND_PER_TILE = CAND_PAD // NS
CAND_CHUNKS = CAND_PER_TILE // CHUNK


def _sc_kernel(constraint, cflat, vflat, w, cand_pad, zeros):
  mesh = plsc.VectorSubcoreMesh(
      core_axis_name="c", subcore_axis_name="s", num_cores=NC, num_subcores=NS)

  @functools.partial(
      pl.kernel,
      mesh=mesh,
      out_type=jax.ShapeDtypeStruct((NC, CAND_PAD, D), jnp.float32),
      scratch_types=[
          pltpu.VMEM((CHUNK,), jnp.int32),
          pltpu.VMEM((CHUNK,), jnp.int32),
          pltpu.VMEM((CHUNK,), jnp.float32),
          pltpu.VMEM((CHUNK, D), jnp.float32),
          pltpu.VMEM_SHARED((N_NODES, D), jnp.float32),
      ],
  )
  def k(constraint_hbm, c_hbm, v_hbm, w_hbm, cand_hbm, zeros_hbm, outp_hbm,
        cidx, vidx, wbuf, rows, acc):
    c = lax.axis_index("c")
    s = lax.axis_index("s")
    wid = c * NS + s

    @pl.when(s < NS - 1)
    def _():
      pltpu.sync_copy(zeros_hbm.at[pl.ds(s * 640, 640)],
                      acc.at[pl.ds(s * 640, 640)])

    @pl.when(s == NS - 1)
    def _():
      pltpu.sync_copy(zeros_hbm.at[pl.ds(9600, 400)],
                      acc.at[pl.ds(9600, 400)])

    plsc.subcore_barrier()

    ebase = wid * EDGES_PER_W

    def chunk_body(kk, _):
      base = ebase + kk * CHUNK
      pltpu.sync_copy(c_hbm.at[pl.ds(base, CHUNK)], cidx)
      pltpu.sync_copy(v_hbm.at[pl.ds(base, CHUNK)], vidx)
      pltpu.sync_copy(w_hbm.at[pl.ds(base, CHUNK)], wbuf)
      pltpu.sync_copy(constraint_hbm.at[cidx], rows)

      def scale_grp(g, _):
        w16 = wbuf[pl.ds(g * 16, 16)]
        for l in range(16):
          wsc = w16[l]
          e = g * 16 + l
          for j in range(D // 16):
            rows[e, pl.ds(j * 16, 16)] = rows[e, pl.ds(j * 16, 16)] * wsc
        return 0

      lax.fori_loop(0, CHUNK // 16, scale_grp, 0)
      pltpu.sync_copy(rows, acc.at[vidx], add=True)
      return 0

    lax.fori_loop(0, N_CHUNKS, chunk_body, 0)
    plsc.subcore_barrier()

    def cand_body(q, _):
      cbase = s * CAND_PER_TILE + q * CHUNK
      pltpu.sync_copy(cand_hbm.at[pl.ds(cbase, CHUNK)], cidx)
      pltpu.sync_copy(acc.at[cidx], rows)
      pltpu.sync_copy(rows, outp_hbm.at[c, pl.ds(cbase, CHUNK)])
      return 0

    lax.fori_loop(0, CAND_CHUNKS, cand_body, 0)

  return k(constraint, cflat, vflat, w, cand_pad, zeros)


def _combine(partials):
  def body(p_ref, o_ref):
    o_ref[...] = p_ref[0] + p_ref[1]

  blk = 256
  return pl.pallas_call(
      body,
      grid=(CAND_PAD // blk,),
      in_specs=[pl.BlockSpec((NC, blk, D), lambda i: (0, i, 0))],
      out_specs=pl.BlockSpec((blk, D), lambda i: (i, 0)),
      out_shape=jax.ShapeDtypeStruct((CAND_PAD, D), jnp.float32),
  )(partials)


def kernel(constraint, variable, cv_edge_index, edge_attr, cand_mask):
  cflat = cv_edge_index[0]
  vflat = cv_edge_index[1]
  w = edge_attr[:, 0]
  cand_pad = jnp.concatenate(
      [cand_mask, jnp.zeros((CAND_PAD - N_CAND,), jnp.int32)])
  zeros = jnp.zeros_like(variable)
  partials = _sc_kernel(constraint, cflat, vflat, w, cand_pad, zeros)
  return _combine(partials)[:N_CAND]


# speedup vs baseline: 9.1445x; 2.0906x over previous
import functools

import jax
import jax.numpy as jnp
from jax import lax
from jax.experimental import pallas as pl
from jax.experimental.pallas import tpu as pltpu
from jax.experimental.pallas import tpu_sc as plsc

N_NODES = 10000
N_EDGES = 320000
D = 128
N_CAND = 5000

NC = 2
NS = 16
NW = NC * NS

EDGES_PER_W = N_EDGES // NW
CHUNK = 80
N_STAGES = 5
SCHUNKS = 25
N_CHUNKS = N_STAGES * SCHUNKS

CAND_PAD = 5120
CAND_PER_TILE = CAND_PAD // NS
CAND_CHUNKS = CAND_PER_TILE // CHUNK


def _sc_kernel(constraint, c4, v4, w4, cand_pad, zeros):
  mesh = plsc.VectorSubcoreMesh(
      core_axis_name="c", subcore_axis_name="s", num_cores=NC, num_subcores=NS)

  @functools.partial(
      pl.kernel,
      mesh=mesh,
      out_type=jax.ShapeDtypeStruct((NC, CAND_PAD, D), jnp.float32),
      scratch_types=[
          pltpu.VMEM((SCHUNKS, CHUNK), jnp.int32),
          pltpu.VMEM((SCHUNKS, CHUNK), jnp.int32),
          pltpu.VMEM((SCHUNKS, CHUNK), jnp.int32),
          pltpu.VMEM((SCHUNKS, CHUNK), jnp.int32),
          pltpu.VMEM((SCHUNKS, CHUNK), jnp.float32),
          pltpu.VMEM((SCHUNKS, CHUNK), jnp.float32),
          pltpu.VMEM((CAND_CHUNKS, CHUNK), jnp.int32),
          pltpu.VMEM((CHUNK, D), jnp.float32),
          pltpu.VMEM((CHUNK, D), jnp.float32),
          pltpu.VMEM_SHARED((N_NODES, D), jnp.float32),
          pltpu.SemaphoreType.DMA,
          pltpu.SemaphoreType.DMA,
          pltpu.SemaphoreType.DMA,
          pltpu.SemaphoreType.DMA,
      ],
  )
  def k(constraint_hbm, c_hbm, v_hbm, w_hbm, cand_hbm, zeros_hbm, outp_hbm,
        cA, cB, vA, vB, wA, wB, q2d, rows0, rows1, acc,
        isemA, isemB, gsem0, gsem1):
    c = lax.axis_index("c")
    s = lax.axis_index("s")
    wid = c * NS + s

    isets = ((cA, vA, wA, isemA), (cB, vB, wB, isemB))
    gbufs = ((rows0, gsem0), (rows1, gsem1))

    pltpu.async_copy(c_hbm.at[wid, 0], cA, isemA)
    pltpu.async_copy(v_hbm.at[wid, 0], vA, isemA)
    pltpu.async_copy(w_hbm.at[wid, 0], wA, isemA)

    @pl.when(s < NS - 1)
    def _():
      pltpu.sync_copy(zeros_hbm.at[pl.ds(s * 640, 640)],
                      acc.at[pl.ds(s * 640, 640)])

    @pl.when(s == NS - 1)
    def _():
      pltpu.sync_copy(zeros_hbm.at[pl.ds(9600, 400)],
                      acc.at[pl.ds(9600, 400)])

    plsc.subcore_barrier()

    def stage_body(t, _):
      for ib in range(2):
        cb, vb, wb, isem = isets[ib]
        ocb, ovb, owb, oisem = isets[1 - ib]

        @pl.when(lax.rem(t, 2) == ib)
        def _(cb=cb, vb=vb, wb=wb, isem=isem,
              ocb=ocb, ovb=ovb, owb=owb, oisem=oisem):
          pltpu.make_async_copy(c_hbm.at[wid, t], cb, isem).wait()
          pltpu.make_async_copy(v_hbm.at[wid, t], vb, isem).wait()
          pltpu.make_async_copy(w_hbm.at[wid, t], wb, isem).wait()

          @pl.when(t + 1 < N_STAGES)
          def _():
            pltpu.async_copy(c_hbm.at[wid, t + 1], ocb, oisem)
            pltpu.async_copy(v_hbm.at[wid, t + 1], ovb, oisem)
            pltpu.async_copy(w_hbm.at[wid, t + 1], owb, oisem)

          pltpu.async_copy(constraint_hbm.at[cb.at[0]], rows0, gsem0)

          def chunk_body(kk, _):
            for b in range(2):
              rbuf, gsem = gbufs[b]
              obuf, ogsem = gbufs[1 - b]

              @pl.when(lax.rem(kk, 2) == b)
              def _(rbuf=rbuf, gsem=gsem, obuf=obuf, ogsem=ogsem):
                pltpu.make_async_copy(
                    constraint_hbm.at[cb.at[kk]], rbuf, gsem).wait()

                @pl.when(kk + 1 < SCHUNKS)
                def _():
                  pltpu.async_copy(
                      constraint_hbm.at[cb.at[kk + 1]], obuf, ogsem)

                def scale_grp(g, _):
                  w16 = wb[kk, pl.ds(g * 16, 16)]
                  for l in range(16):
                    wsc = w16[l]
                    e = g * 16 + l
                    for j in range(D // 16):
                      rbuf[e, pl.ds(j * 16, 16)] = (
                          rbuf[e, pl.ds(j * 16, 16)] * wsc)
                  return 0

                lax.fori_loop(0, CHUNK // 16, scale_grp, 0)

                pltpu.sync_copy(rbuf, acc.at[vb.at[kk]], add=True)

            return 0

          lax.fori_loop(0, SCHUNKS, chunk_body, 0)

      return 0

    lax.fori_loop(0, N_STAGES, stage_body, 0)
    plsc.subcore_barrier()

    pltpu.sync_copy(cand_hbm.at[s], q2d)
    for q in range(CAND_CHUNKS):
      rbuf, sem = gbufs[q % 2]
      cbase = s * CAND_PER_TILE + q * CHUNK
      if q >= 2:
        pbase = s * CAND_PER_TILE + (q - 2) * CHUNK
        pltpu.make_async_copy(
            rbuf, outp_hbm.at[c, pl.ds(pbase, CHUNK)], sem).wait()
      pltpu.sync_copy(acc.at[q2d.at[q]], rbuf)
      pltpu.async_copy(rbuf, outp_hbm.at[c, pl.ds(cbase, CHUNK)], sem)
    for q in range(CAND_CHUNKS - 2, CAND_CHUNKS):
      rbuf, sem = gbufs[q % 2]
      cbase = s * CAND_PER_TILE + q * CHUNK
      pltpu.make_async_copy(
          rbuf, outp_hbm.at[c, pl.ds(cbase, CHUNK)], sem).wait()

  return k(constraint, c4, v4, w4, cand_pad, zeros)


def _combine(partials):
  def body(p_ref, o_ref):
    o_ref[...] = p_ref[0] + p_ref[1]

  blk = 256
  return pl.pallas_call(
      body,
      grid=(CAND_PAD // blk,),
      in_specs=[pl.BlockSpec((NC, blk, D), lambda i: (0, i, 0))],
      out_specs=pl.BlockSpec((blk, D), lambda i: (i, 0)),
      out_shape=jax.ShapeDtypeStruct((CAND_PAD, D), jnp.float32),
  )(partials)


def kernel(constraint, variable, cv_edge_index, edge_attr, cand_mask):
  c4 = cv_edge_index[0].reshape(NW, N_STAGES, SCHUNKS, CHUNK)
  v4 = cv_edge_index[1].reshape(NW, N_STAGES, SCHUNKS, CHUNK)
  w4 = edge_attr[:, 0].reshape(NW, N_STAGES, SCHUNKS, CHUNK)
  cand_pad = jnp.concatenate(
      [cand_mask, jnp.zeros((CAND_PAD - N_CAND,), jnp.int32)]
  ).reshape(NS, CAND_CHUNKS, CHUNK)
  zeros = jnp.zeros_like(variable)
  partials = _sc_kernel(constraint, c4, v4, w4, cand_pad, zeros)
  return _combine(partials)[:N_CAND]


# speedup vs baseline: 9.3827x; 1.0260x over previous
import functools

import jax
import jax.numpy as jnp
from jax import lax
from jax.experimental import pallas as pl
from jax.experimental.pallas import tpu as pltpu
from jax.experimental.pallas import tpu_sc as plsc

N_NODES = 10000
N_EDGES = 320000
D = 128
N_CAND = 5000

NC = 2
NS = 16
NW = NC * NS

EDGES_PER_W = N_EDGES // NW
CHUNK = 80
N_STAGES = 5
SCHUNKS = 25
N_CHUNKS = N_STAGES * SCHUNKS
EPS = EDGES_PER_W // N_STAGES

CAND_PAD = 5120
CAND_PER_TILE = CAND_PAD // NS
CAND_CHUNKS = CAND_PER_TILE // CHUNK


def _sc_kernel(constraint, cflat, v4, wflat, cand_pad, zeros):
  mesh = plsc.VectorSubcoreMesh(
      core_axis_name="c", subcore_axis_name="s", num_cores=NC, num_subcores=NS)

  @functools.partial(
      pl.kernel,
      mesh=mesh,
      out_type=jax.ShapeDtypeStruct((NC, CAND_PAD, D), jnp.float32),
      scratch_types=[
          pltpu.VMEM((2 * 2048,), jnp.int32),
          pltpu.VMEM((2, SCHUNKS, CHUNK), jnp.int32),
          pltpu.VMEM((2 * 2048,), jnp.float32),
          pltpu.VMEM((CAND_CHUNKS, CHUNK), jnp.int32),
          pltpu.VMEM((CHUNK, D), jnp.float32),
          pltpu.VMEM((CHUNK, D), jnp.float32),
          pltpu.VMEM((CHUNK, D), jnp.float32),
          pltpu.VMEM_SHARED((N_NODES, D), jnp.float32),
          pltpu.SemaphoreType.DMA,
          pltpu.SemaphoreType.DMA,
          pltpu.SemaphoreType.DMA,
          pltpu.SemaphoreType.DMA,
          pltpu.SemaphoreType.DMA,
          pltpu.SemaphoreType.DMA,
          pltpu.SemaphoreType.DMA,
      ],
  )
  def k(constraint_hbm, c_hbm, v_hbm, w_hbm, cand_hbm, zeros_hbm, outp_hbm,
        cb, vb, wb, q2d, rows0, rows1, rows2, acc,
        isem, gsem0, gsem1, gsem2, ssem0, ssem1, ssem2):
    c = lax.axis_index("c")
    s = lax.axis_index("s")
    wid = c * NS + s
    ebase = wid * EDGES_PER_W

    gbufs = ((rows0, gsem0, ssem0), (rows1, gsem1, ssem1),
             (rows2, gsem2, ssem2))

    def ioff(ib):
      return pl.multiple_of(ib * 2048, 8)

    def stage_copies(t, ib):
      off = pl.multiple_of(ebase + t * EPS, 8)
      pltpu.async_copy(c_hbm.at[pl.ds(off, EPS)], cb.at[pl.ds(ioff(ib), EPS)],
                       isem)
      pltpu.async_copy(v_hbm.at[wid, t], vb.at[ib], isem)
      pltpu.async_copy(w_hbm.at[pl.ds(off, EPS)], wb.at[pl.ds(ioff(ib), EPS)],
                       isem)

    def stage_waits(t, ib):
      off = pl.multiple_of(ebase + t * EPS, 8)
      pltpu.make_async_copy(c_hbm.at[pl.ds(off, EPS)],
                            cb.at[pl.ds(ioff(ib), EPS)], isem).wait()
      pltpu.make_async_copy(v_hbm.at[wid, t], vb.at[ib], isem).wait()
      pltpu.make_async_copy(w_hbm.at[pl.ds(off, EPS)],
                            wb.at[pl.ds(ioff(ib), EPS)], isem).wait()

    stage_copies(0, 0)

    @pl.when(s < NS - 1)
    def _():
      pltpu.sync_copy(zeros_hbm.at[pl.ds(s * 640, 640)],
                      acc.at[pl.ds(s * 640, 640)])

    @pl.when(s == NS - 1)
    def _():
      pltpu.sync_copy(zeros_hbm.at[pl.ds(9600, 400)],
                      acc.at[pl.ds(9600, 400)])

    stage_waits(0, 0)
    pltpu.async_copy(
        constraint_hbm.at[cb.at[pl.ds(0, CHUNK)]], rows0, gsem0)

    plsc.subcore_barrier()

    def chunk_body(kk, _):
      t = lax.div(kk, SCHUNKS)
      r = lax.rem(kk, SCHUNKS)
      ib = lax.rem(t, 2)

      @pl.when((r == 0) & (t + 1 < N_STAGES))
      def _():
        stage_copies(t + 1, lax.rem(t + 1, 2))

      for b in range(3):
        rbuf, gsem, ssem = gbufs[b]
        nb = (b + 1) % 3
        nbuf, ngsem, nssem = gbufs[nb]

        @pl.when(lax.rem(kk, 3) == b)
        def _(rbuf=rbuf, gsem=gsem, ssem=ssem,
              nbuf=nbuf, ngsem=ngsem, nssem=nssem):
          coff = pl.multiple_of(ioff(ib) + r * CHUNK, 8)
          pltpu.make_async_copy(
              constraint_hbm.at[cb.at[pl.ds(coff, CHUNK)]],
              rbuf, gsem).wait()

          @pl.when(kk >= 2)
          def _():
            pltpu.make_async_copy(
                nbuf, acc.at[vb.at[ib, r]], nssem).wait()

          @pl.when(kk + 1 < N_CHUNKS)
          def _():
            kk1 = kk + 1
            t1 = lax.div(kk1, SCHUNKS)
            r1 = lax.rem(kk1, SCHUNKS)
            ib1 = lax.rem(t1, 2)

            @pl.when(r1 == 0)
            def _():
              stage_waits(t1, ib1)

            coff1 = pl.multiple_of(ioff(ib1) + r1 * CHUNK, 8)
            pltpu.async_copy(
                constraint_hbm.at[cb.at[pl.ds(coff1, CHUNK)]],
                nbuf, ngsem)

          def scale_grp(g, _):
            w16 = wb[pl.ds(ioff(ib) + r * CHUNK + g * 16, 16)]
            for l in range(16):
              wsc = w16[l]
              e = g * 16 + l
              for j in range(D // 16):
                rbuf[e, pl.ds(j * 16, 16)] = rbuf[e, pl.ds(j * 16, 16)] * wsc
            return 0

          lax.fori_loop(0, CHUNK // 16, scale_grp, 0)

          pltpu.async_copy(rbuf, acc.at[vb.at[ib, r]], ssem, add=True)

      return 0

    lax.fori_loop(0, N_CHUNKS, chunk_body, 0)

    pltpu.make_async_copy(rows0, acc.at[vb.at[1, 24]], ssem0).wait()
    pltpu.make_async_copy(rows1, acc.at[vb.at[1, 24]], ssem1).wait()
    plsc.subcore_barrier()

    pltpu.sync_copy(cand_hbm.at[s], q2d)
    for q in range(CAND_CHUNKS):
      rbuf, sem, _ = gbufs[q % 2]
      cbase = s * CAND_PER_TILE + q * CHUNK
      if q >= 2:
        pbase = s * CAND_PER_TILE + (q - 2) * CHUNK
        pltpu.make_async_copy(
            rbuf, outp_hbm.at[c, pl.ds(pbase, CHUNK)], sem).wait()
      pltpu.sync_copy(acc.at[q2d.at[q]], rbuf)
      pltpu.async_copy(rbuf, outp_hbm.at[c, pl.ds(cbase, CHUNK)], sem)
    for q in range(CAND_CHUNKS - 2, CAND_CHUNKS):
      rbuf, sem, _ = gbufs[q % 2]
      cbase = s * CAND_PER_TILE + q * CHUNK
      pltpu.make_async_copy(
          rbuf, outp_hbm.at[c, pl.ds(cbase, CHUNK)], sem).wait()

  return k(constraint, cflat, v4, wflat, cand_pad, zeros)


def _combine(partials):
  def body(p_ref, o_ref):
    o_ref[...] = p_ref[0] + p_ref[1]

  blk = 256
  return pl.pallas_call(
      body,
      grid=(CAND_PAD // blk,),
      in_specs=[pl.BlockSpec((NC, blk, D), lambda i: (0, i, 0))],
      out_specs=pl.BlockSpec((blk, D), lambda i: (i, 0)),
      out_shape=jax.ShapeDtypeStruct((CAND_PAD, D), jnp.float32),
  )(partials)


def kernel(constraint, variable, cv_edge_index, edge_attr, cand_mask):
  cflat = cv_edge_index[0]
  v4 = cv_edge_index[1].reshape(NW, N_STAGES, SCHUNKS, CHUNK)
  wflat = edge_attr[:, 0]
  cand_pad = jnp.concatenate(
      [cand_mask, jnp.zeros((CAND_PAD - N_CAND,), jnp.int32)]
  ).reshape(NS, CAND_CHUNKS, CHUNK)
  zeros = jnp.zeros_like(variable)
  partials = _sc_kernel(constraint, cflat, v4, wflat, cand_pad, zeros)
  return _combine(partials)[:N_CAND]


# speedup vs baseline: 10.0564x; 1.0718x over previous
import functools

import jax
import jax.numpy as jnp
from jax import lax
from jax.experimental import pallas as pl
from jax.experimental.pallas import tpu as pltpu
from jax.experimental.pallas import tpu_sc as plsc

N_NODES = 10000
N_EDGES = 320000
D = 128
N_CAND = 5000

NC = 2
NS = 16
NW = NC * NS

EDGES_PER_W = N_EDGES // NW
CHUNK = 80
N_STAGES = 5
SCHUNKS = 25
N_CHUNKS = N_STAGES * SCHUNKS
EPS = EDGES_PER_W // N_STAGES

CAND_PAD = 5120
CAND_PER_TILE = CAND_PAD // NS
CAND_CHUNKS = CAND_PER_TILE // CHUNK


def _sc_kernel(constraint, cflat, vflat, wflat, cand_pad, zeros):
  mesh = plsc.VectorSubcoreMesh(
      core_axis_name="c", subcore_axis_name="s", num_cores=NC, num_subcores=NS)

  @functools.partial(
      pl.kernel,
      mesh=mesh,
      out_type=jax.ShapeDtypeStruct((NC, CAND_PAD, D), jnp.float32),
      scratch_types=[
          pltpu.VMEM((2 * 2048,), jnp.int32),
          pltpu.VMEM((3, 1, CHUNK), jnp.int32),
          pltpu.VMEM((2 * 2048,), jnp.float32),
          pltpu.VMEM((CAND_CHUNKS, CHUNK), jnp.int32),
          pltpu.VMEM((CHUNK, D), jnp.float32),
          pltpu.VMEM((CHUNK, D), jnp.float32),
          pltpu.VMEM((CHUNK, D), jnp.float32),
          pltpu.VMEM_SHARED((N_NODES, D), jnp.float32),
          pltpu.SemaphoreType.DMA,
          pltpu.SemaphoreType.DMA,
          pltpu.SemaphoreType.DMA,
          pltpu.SemaphoreType.DMA,
          pltpu.SemaphoreType.DMA,
          pltpu.SemaphoreType.DMA,
          pltpu.SemaphoreType.DMA,
      ],
  )
  def k(constraint_hbm, c_hbm, v_hbm, w_hbm, cand_hbm, zeros_hbm, outp_hbm,
        cb, vst, wb, q2d, rows0, rows1, rows2, acc,
        isem, gsem0, gsem1, gsem2, ssem0, ssem1, ssem2):
    c = lax.axis_index("c")
    s = lax.axis_index("s")
    wid = c * NS + s
    ebase = wid * EDGES_PER_W

    gbufs = ((rows0, gsem0, ssem0), (rows1, gsem1, ssem1),
             (rows2, gsem2, ssem2))

    def ioff(ib):
      return pl.multiple_of(ib * 2048, 8)

    def stage_copies(t, ib):
      off = pl.multiple_of(ebase + t * EPS, 8)
      pltpu.async_copy(c_hbm.at[pl.ds(off, EPS)],
                       cb.at[pl.ds(ioff(ib), EPS)], isem)
      pltpu.async_copy(w_hbm.at[pl.ds(off, EPS)], wb.at[pl.ds(ioff(ib), EPS)],
                       isem)

    def stage_waits(t, ib):
      off = pl.multiple_of(ebase + t * EPS, 8)
      pltpu.make_async_copy(c_hbm.at[pl.ds(off, EPS)],
                            cb.at[pl.ds(ioff(ib), EPS)], isem).wait()
      pltpu.make_async_copy(w_hbm.at[pl.ds(off, EPS)],
                            wb.at[pl.ds(ioff(ib), EPS)], isem).wait()

    def vcopy(kk1, slot):
      voff = pl.multiple_of(ebase + kk1 * CHUNK, 8)
      return v_hbm.at[pl.ds(voff, CHUNK)], vst.at[slot, 0]

    stage_copies(0, 0)

    @pl.when(s < NS - 1)
    def _():
      pltpu.sync_copy(zeros_hbm.at[pl.ds(s * 640, 640)],
                      acc.at[pl.ds(s * 640, 640)])

    @pl.when(s == NS - 1)
    def _():
      pltpu.sync_copy(zeros_hbm.at[pl.ds(9600, 400)],
                      acc.at[pl.ds(9600, 400)])

    stage_waits(0, 0)
    vsrc0, vdst0 = vcopy(0, 0)
    pltpu.async_copy(vsrc0, vdst0, gsem0)
    pltpu.async_copy(
        constraint_hbm.at[cb.at[pl.ds(0, CHUNK)]], rows0, gsem0)

    plsc.subcore_barrier()

    def chunk_body(kk, _):
      t = lax.div(kk, SCHUNKS)
      r = lax.rem(kk, SCHUNKS)
      ib = lax.rem(t, 2)

      @pl.when((r == 0) & (t + 1 < N_STAGES))
      def _():
        stage_copies(t + 1, lax.rem(t + 1, 2))

      for b in range(3):
        rbuf, gsem, ssem = gbufs[b]
        nb = (b + 1) % 3
        nbuf, ngsem, nssem = gbufs[nb]

        @pl.when(lax.rem(kk, 3) == b)
        def _(rbuf=rbuf, gsem=gsem, ssem=ssem,
              nbuf=nbuf, ngsem=ngsem, nssem=nssem):
          vsrc, vdst = vcopy(kk, lax.rem(kk, 3))
          pltpu.make_async_copy(vsrc, vdst, gsem).wait()
          coff = pl.multiple_of(ioff(ib) + r * CHUNK, 8)
          pltpu.make_async_copy(
              constraint_hbm.at[cb.at[pl.ds(coff, CHUNK)]],
              rbuf, gsem).wait()

          @pl.when(kk >= 2)
          def _():
            pltpu.make_async_copy(
                nbuf, acc.at[vst.at[0, 0]], nssem).wait()

          @pl.when(kk + 1 < N_CHUNKS)
          def _():
            kk1 = kk + 1
            t1 = lax.div(kk1, SCHUNKS)
            r1 = lax.rem(kk1, SCHUNKS)
            ib1 = lax.rem(t1, 2)

            @pl.when(r1 == 0)
            def _():
              stage_waits(t1, ib1)

            vsrc1, vdst1 = vcopy(kk1, lax.rem(kk1, 3))
            pltpu.async_copy(vsrc1, vdst1, ngsem)
            coff1 = pl.multiple_of(ioff(ib1) + r1 * CHUNK, 8)
            pltpu.async_copy(
                constraint_hbm.at[cb.at[pl.ds(coff1, CHUNK)]],
                nbuf, ngsem)

          def scale_grp(g, _):
            w16 = wb[pl.ds(ioff(ib) + r * CHUNK + g * 16, 16)]
            for l in range(16):
              wsc = w16[l]
              e = g * 16 + l
              for j in range(D // 16):
                rbuf[e, pl.ds(j * 16, 16)] = rbuf[e, pl.ds(j * 16, 16)] * wsc
            return 0

          lax.fori_loop(0, CHUNK // 16, scale_grp, 0)

          pltpu.async_copy(
              rbuf, acc.at[vst.at[lax.rem(kk, 3), 0]], ssem, add=True)

      return 0

    lax.fori_loop(0, N_CHUNKS, chunk_body, 0)

    pltpu.make_async_copy(rows0, acc.at[vst.at[0, 0]], ssem0).wait()
    pltpu.make_async_copy(rows1, acc.at[vst.at[0, 0]], ssem1).wait()
    plsc.subcore_barrier()

    pltpu.sync_copy(cand_hbm.at[s], q2d)
    for q in range(CAND_CHUNKS):
      rbuf, sem, _ = gbufs[q % 2]
      cbase = s * CAND_PER_TILE + q * CHUNK
      if q >= 2:
        pbase = s * CAND_PER_TILE + (q - 2) * CHUNK
        pltpu.make_async_copy(
            rbuf, outp_hbm.at[c, pl.ds(pbase, CHUNK)], sem).wait()
      pltpu.sync_copy(acc.at[q2d.at[q]], rbuf)
      pltpu.async_copy(rbuf, outp_hbm.at[c, pl.ds(cbase, CHUNK)], sem)
    for q in range(CAND_CHUNKS - 2, CAND_CHUNKS):
      rbuf, sem, _ = gbufs[q % 2]
      cbase = s * CAND_PER_TILE + q * CHUNK
      pltpu.make_async_copy(
          rbuf, outp_hbm.at[c, pl.ds(cbase, CHUNK)], sem).wait()

  return k(constraint, cflat, vflat, wflat, cand_pad, zeros)


def _combine(partials):
  def body(p_ref, o_ref):
    o_ref[...] = p_ref[0] + p_ref[1]

  blk = 1000
  return pl.pallas_call(
      body,
      grid=(N_CAND // blk,),
      in_specs=[pl.BlockSpec((NC, blk, D), lambda i: (0, i, 0))],
      out_specs=pl.BlockSpec((blk, D), lambda i: (i, 0)),
      out_shape=jax.ShapeDtypeStruct((N_CAND, D), jnp.float32),
  )(partials)


def kernel(constraint, variable, cv_edge_index, edge_attr, cand_mask):
  cflat = cv_edge_index[0]
  vflat = cv_edge_index[1]
  wflat = edge_attr[:, 0]
  cand_pad = jnp.concatenate(
      [cand_mask, jnp.zeros((CAND_PAD - N_CAND,), jnp.int32)]
  ).reshape(NS, CAND_CHUNKS, CHUNK)
  zeros = jnp.zeros_like(variable)
  partials = _sc_kernel(constraint, cflat, vflat, wflat, cand_pad, zeros)
  return _combine(partials)
